# baseline (speedup 1.0000x reference)
import jax
import jax.numpy as jnp
from jax.experimental import pallas as pl


_TILE = 256


def _embed_kernel(x_ref, fe_ref, w_ref, inst_ref, supi_ref, supf_ref):
    i = pl.program_id(0)
    feat = fe_ref[...]
    inst = jnp.dot(x_ref[...], feat, preferred_element_type=jnp.float32)
    inst_ref[...] = inst
    supi_ref[...] = jnp.dot(inst, w_ref[...], preferred_element_type=jnp.float32)

    @pl.when(i == 0)
    def _():
        supf_ref[...] = jnp.dot(feat, w_ref[...], preferred_element_type=jnp.float32)


def _gcn_kernel(adj_ref, supf_ref, supi_ref, w1_ref, b1_ref, w2_ref, b2_ref,
                logits_ref, lat_ref):
    f = supf_ref.shape[0]
    a = adj_ref[...]
    lat = jnp.dot(a[:, :f], supf_ref[...], preferred_element_type=jnp.float32)
    lat = lat + jnp.dot(a[:, f:], supi_ref[...], preferred_element_type=jnp.float32)
    lat = jnp.maximum(lat, 0.0)
    lat_ref[...] = lat
    h = jnp.maximum(
        jnp.dot(lat, w1_ref[...], preferred_element_type=jnp.float32) + b1_ref[...],
        0.0)
    logits_ref[...] = (
        jnp.dot(h, w2_ref[...], preferred_element_type=jnp.float32) + b2_ref[...])


def kernel(X_batch, adj, n_curr_features, feature_embeddings, gcn_weight,
           W1, b1, W2, b2):
    B, F = X_batch.shape
    H = gcn_weight.shape[0]
    C = W2.shape[1]
    Hh = W1.shape[1]
    T = _TILE
    grid = (B // T,)

    inst, sup_i, sup_f = pl.pallas_call(
        _embed_kernel,
        grid=grid,
        in_specs=[
            pl.BlockSpec((T, F), lambda i: (i, 0)),
            pl.BlockSpec((F, H), lambda i: (0, 0)),
            pl.BlockSpec((H, H), lambda i: (0, 0)),
        ],
        out_specs=[
            pl.BlockSpec((T, H), lambda i: (i, 0)),
            pl.BlockSpec((T, H), lambda i: (i, 0)),
            pl.BlockSpec((F, H), lambda i: (0, 0)),
        ],
        out_shape=[
            jax.ShapeDtypeStruct((B, H), jnp.float32),
            jax.ShapeDtypeStruct((B, H), jnp.float32),
            jax.ShapeDtypeStruct((F, H), jnp.float32),
        ],
    )(X_batch, feature_embeddings, gcn_weight)

    nblk = F // T
    logits, latent = pl.pallas_call(
        _gcn_kernel,
        grid=grid,
        in_specs=[
            pl.BlockSpec((T, F + B), lambda i: (i + nblk, 0)),
            pl.BlockSpec((F, H), lambda i: (0, 0)),
            pl.BlockSpec((B, H), lambda i: (0, 0)),
            pl.BlockSpec((H, Hh), lambda i: (0, 0)),
            pl.BlockSpec((1, Hh), lambda i: (0, 0)),
            pl.BlockSpec((Hh, C), lambda i: (0, 0)),
            pl.BlockSpec((1, C), lambda i: (0, 0)),
        ],
        out_specs=[
            pl.BlockSpec((T, C), lambda i: (i, 0)),
            pl.BlockSpec((T, H), lambda i: (i, 0)),
        ],
        out_shape=[
            jax.ShapeDtypeStruct((B, C), jnp.float32),
            jax.ShapeDtypeStruct((B, H), jnp.float32),
        ],
    )(adj, sup_f, sup_i, W1, b1.reshape(1, Hh), W2, b2.reshape(1, C))

    return (logits, latent, inst)
